# Initial kernel scaffold; baseline (speedup 1.0000x reference)
import jax
import jax.numpy as jnp
from jax.experimental import pallas as pl


def kernel(x, params, edge_index, batch):
    raise NotImplementedError("write your pallas kernel here")


# speedup vs baseline: 14.3449x; 14.3449x over previous
import functools

import jax
import jax.numpy as jnp
from jax import lax
from jax.experimental import pallas as pl
from jax.experimental.pallas import tpu as pltpu
from jax.experimental.pallas import tpu_sc as plsc

N = 10000
E = 320000
F = 128
D = 256
OUT = 40
G = 64
NUM_LAYERS = 4
QUAR = D // 4

NC = 2
NS = 16
EB = 80
EBW = 256
NB_MAIN = 78
NB_TAIL = 80
E_MAIN = (NS - 1) * NB_MAIN * EBW
DPT = E // (NC * NS)
DNB = DPT // EB
RPT = 624
RPT_LAST = N - (NS - 1) * RPT

@functools.lru_cache(maxsize=None)
def _mesh():
    return plsc.VectorSubcoreMesh(
        core_axis_name="c", subcore_axis_name="s",
        num_cores=NC, num_subcores=NS)


def _row_split(s, fn):
    base = pl.multiple_of(s * RPT, 8)

    @pl.when(s < NS - 1)
    def _():
        fn(base, RPT)

    @pl.when(s == NS - 1)
    def _():
        fn((NS - 1) * RPT, RPT_LAST)


def _deg_body(col3, ones_hbm, zeros_hbm, d0, d1, col_all, ones_v, acc, sem):
    c = lax.axis_index("c")
    s = lax.axis_index("s")
    w = c * NS + s
    _row_split(s, lambda base, n:
               pltpu.sync_copy(zeros_hbm.at[pl.ds(0, n)],
                               acc.at[pl.ds(base, n)]))
    pltpu.sync_copy(ones_hbm, ones_v)
    pltpu.sync_copy(col3.at[w], col_all)
    plsc.subcore_barrier()

    def body(i, carry):
        pltpu.sync_copy(ones_v, acc.at[col_all.at[i]], add=True)
        return carry

    lax.fori_loop(0, DNB, body, 0, unroll=False)
    plsc.subcore_barrier()

    @pl.when(c == 0)
    def _():
        _row_split(s, lambda base, n:
                   pltpu.sync_copy(acc.at[pl.ds(base, n)],
                                   d0.at[pl.ds(base, n)]))

    @pl.when(c == 1)
    def _():
        _row_split(s, lambda base, n:
                   pltpu.sync_copy(acc.at[pl.ds(base, n)],
                                   d1.at[pl.ds(base, n)]))


@functools.lru_cache(maxsize=None)
def _deg_kernel():
    return pl.kernel(
        _deg_body,
        out_type=[jax.ShapeDtypeStruct((N, 16), jnp.float32),
                  jax.ShapeDtypeStruct((N, 16), jnp.float32)],
        mesh=_mesh(),
        scratch_types=[
            pltpu.VMEM((DNB, EB), jnp.int32),
            pltpu.VMEM((EB, 16), jnp.float32),
            pltpu.VMEM_SHARED((N, 16), jnp.float32),
            pltpu.SemaphoreType.DMA,
        ],
        compiler_params=pltpu.CompilerParams(use_tc_tiling_on_sc=False),
    )


def _edge_body(hs0, hs1, hs2, hs3, row_m, row_t, col_m, col_t, t0, t1, t2, t3,
               row_all, col_all, buf0, buf1, buf2, acc,
               gsem0, gsem1, gsem2, ssem0, ssem1, ssem2):
    c = lax.axis_index("c")
    s = lax.axis_index("s")

    @pl.when(s < NS - 1)
    def _():
        pltpu.sync_copy(row_m.at[s], row_all.at[pl.ds(0, NB_MAIN)])
        pltpu.sync_copy(col_m.at[s], col_all.at[pl.ds(0, NB_MAIN)])

    @pl.when(s == NS - 1)
    def _():
        pltpu.sync_copy(row_t, row_all)
        pltpu.sync_copy(col_t, col_all)

    nb = lax.select(s == NS - 1, NB_TAIL, NB_MAIN)

    def run(tab, out):
        _row_split(s, lambda base, n:
                   pltpu.sync_copy(tab.at[pl.ds(base, n)],
                                   acc.at[pl.ds(base, n)]))
        plsc.subcore_barrier()

        bufs = (buf0, buf1, buf2)
        gsems = (gsem0, gsem1, gsem2)
        ssems = (ssem0, ssem1, ssem2)
        pltpu.async_copy(tab.at[row_all.at[0]], buf0, gsem0)

        def step(i, j):
            nj = (j + 1) % 3
            pltpu.make_async_copy(tab.at[row_all.at[i]], bufs[j], gsems[j]).wait()

            @pl.when(i >= 2)
            def _():
                pltpu.make_async_copy(
                    bufs[nj], acc.at[col_all.at[i - 2]], ssems[nj]).wait()

            @pl.when(i + 1 < nb)
            def _():
                pltpu.async_copy(tab.at[row_all.at[i + 1]], bufs[nj], gsems[nj])

            pltpu.async_copy(bufs[j], acc.at[col_all.at[i]], ssems[j], add=True)

        def body(i, carry):
            for j in range(3):
                @pl.when(lax.rem(i, 3) == j)
                def _(j=j):
                    step(i, j)

            return carry

        lax.fori_loop(0, nb, body, 0, unroll=False)

        for k in (nb - 2, nb - 1):
            for j in range(3):
                @pl.when(lax.rem(k, 3) == j)
                def _(k=k, j=j):
                    pltpu.make_async_copy(
                        bufs[j], acc.at[col_all.at[k]], ssems[j]).wait()
        plsc.subcore_barrier()
        _row_split(s, lambda base, n:
                   pltpu.sync_copy(acc.at[pl.ds(base, n)],
                                   out.at[pl.ds(base, n)]))

    for cc, tab, out in ((0, hs0, t0), (1, hs1, t1), (0, hs2, t2),
                         (1, hs3, t3)):
        @pl.when(c == cc)
        def _(tab=tab, out=out):
            run(tab, out)


@functools.lru_cache(maxsize=None)
def _edge_kernel():
    return pl.kernel(
        _edge_body,
        out_type=[jax.ShapeDtypeStruct((N, QUAR), jnp.float32)
                  for _ in range(4)],
        mesh=_mesh(),
        scratch_types=[
            pltpu.VMEM((NB_TAIL, EBW), jnp.int32),
            pltpu.VMEM((NB_TAIL, EBW), jnp.int32),
            pltpu.VMEM((EBW, QUAR), jnp.float32),
            pltpu.VMEM((EBW, QUAR), jnp.float32),
            pltpu.VMEM((EBW, QUAR), jnp.float32),
            pltpu.VMEM_SHARED((N, QUAR), jnp.float32),
            pltpu.SemaphoreType.DMA,
            pltpu.SemaphoreType.DMA,
            pltpu.SemaphoreType.DMA,
            pltpu.SemaphoreType.DMA,
            pltpu.SemaphoreType.DMA,
            pltpu.SemaphoreType.DMA,
        ],
        compiler_params=pltpu.CompilerParams(use_tc_tiling_on_sc=False),
    )


def _pt_body(batch_ref, pt_ref):
    b = batch_ref[...]
    seg = lax.broadcasted_iota(jnp.int32, (G, N), 0)
    pt_ref[...] = (seg == b).astype(jnp.float32)


def _dinv(d0_ref, d1_ref):
    deg = d0_ref[...][:, :1] + d1_ref[...][:, :1] + 1.0
    return lax.rsqrt(deg)


def _pre_core(h, w1_ref, b1_ref, g1_ref, bb1_ref, w2_ref, b2_ref, wg_ref,
              d0_ref, d1_ref, hs_refs):
    y1 = jnp.dot(h, w1_ref[...], preferred_element_type=jnp.float32) + b1_ref[...]
    m = jnp.mean(y1, axis=0)
    v = jnp.mean(y1 * y1, axis=0) - m * m
    a = jnp.maximum(
        (y1 - m) * lax.rsqrt(v + 1e-5) * g1_ref[...] + bb1_ref[...], 0.0)
    w2g = jnp.dot(w2_ref[...], wg_ref[...], preferred_element_type=jnp.float32)
    bg = jnp.dot(b2_ref[...].reshape(1, -1), wg_ref[...],
                 preferred_element_type=jnp.float32)
    hg = jnp.dot(a, w2g, preferred_element_type=jnp.float32) + bg
    hs = hg * _dinv(d0_ref, d1_ref)
    for k, ref in enumerate(hs_refs):
        ref[...] = hs[:, k * QUAR:(k + 1) * QUAR]


def _pre_body(h_ref, w1_ref, b1_ref, g1_ref, bb1_ref, w2_ref, b2_ref, wg_ref,
              d0_ref, d1_ref, *hs_refs):
    _pre_core(h_ref[...], w1_ref, b1_ref, g1_ref, bb1_ref, w2_ref, b2_ref,
              wg_ref, d0_ref, d1_ref, hs_refs)


def _post_core(t0_ref, t1_ref, t2_ref, t3_ref, d0_ref, d1_ref, gb_ref,
               bng_ref, bnb_ref):
    t = jnp.concatenate(
        [t0_ref[...], t1_ref[...], t2_ref[...], t3_ref[...]], axis=1)
    conv = t * _dinv(d0_ref, d1_ref) + gb_ref[...]
    m = jnp.mean(conv, axis=0)
    v = jnp.mean(conv * conv, axis=0) - m * m
    return jnp.maximum(
        (conv - m) * lax.rsqrt(v + 1e-5) * bng_ref[...] + bnb_ref[...], 0.0)


def _post_body(t0_ref, t1_ref, t2_ref, t3_ref, d0_ref, d1_ref, gb_ref,
               bng_ref, bnb_ref, pt_ref, hn_ref, pooled_ref):
    hn = _post_core(t0_ref, t1_ref, t2_ref, t3_ref, d0_ref, d1_ref, gb_ref,
                    bng_ref, bnb_ref)
    hn_ref[...] = hn
    pooled_ref[...] = jnp.dot(pt_ref[...], hn,
                              preferred_element_type=jnp.float32)


def _post_last_body(t0_ref, t1_ref, t2_ref, t3_ref, d0_ref, d1_ref, gb_ref,
                    bng_ref, bnb_ref, pt_ref, pooled_ref):
    hn = _post_core(t0_ref, t1_ref, t2_ref, t3_ref, d0_ref, d1_ref, gb_ref,
                    bng_ref, bnb_ref)
    pooled_ref[...] = jnp.dot(pt_ref[...], hn,
                              preferred_element_type=jnp.float32)


def _final_body(pt_ref, x_ref, p1, p2, p3, p4, w0, w1, w2, w3, w4,
                c0, c1, c2, c3, c4, out_ref):
    p0 = jnp.dot(pt_ref[...], x_ref[...], preferred_element_type=jnp.float32)
    acc = jnp.dot(p0, w0[...], preferred_element_type=jnp.float32) + c0[...]
    for p, w, cb in ((p1, w1, c1), (p2, w2, c2), (p3, w3, c3), (p4, w4, c4)):
        acc = acc + jnp.dot(p[...], w[...],
                            preferred_element_type=jnp.float32) + cb[...]
    out_ref[...] = acc


def _tc_call(body, out_shape):
    return pl.pallas_call(
        body, out_shape=out_shape,
        compiler_params=pltpu.CompilerParams(
            vmem_limit_bytes=100 * 1024 * 1024))


def kernel(x, params, edge_index, batch):
    row = edge_index[0].astype(jnp.int32)
    col = edge_index[1].astype(jnp.int32)
    row_m = row[:E_MAIN].reshape(NS - 1, NB_MAIN, EBW)
    row_t = row[E_MAIN:].reshape(NB_TAIL, EBW)
    col_m = col[:E_MAIN].reshape(NS - 1, NB_MAIN, EBW)
    col_t = col[E_MAIN:].reshape(NB_TAIL, EBW)
    col3d = col.reshape(NC * NS, DNB, EB)
    ones16 = jnp.ones((EB, 16), jnp.float32)
    zeros16 = jnp.zeros((RPT_LAST, 16), jnp.float32)

    pt = _tc_call(_pt_body, jax.ShapeDtypeStruct((G, N), jnp.float32))(
        batch.astype(jnp.int32).reshape(1, N))

    d0, d1 = _deg_kernel()(col3d, ones16, zeros16)

    hs_shapes = [jax.ShapeDtypeStruct((N, QUAR), jnp.float32)
                 for _ in range(4)]
    pooled = []
    h = x
    for i in range(NUM_LAYERS):
        hsq = _tc_call(_pre_body, hs_shapes)(
            h, params[f"l1W{i}"], params[f"l1b{i}"], params[f"bn1g{i}"],
            params[f"bn1b{i}"], params[f"l2W{i}"], params[f"l2b{i}"],
            params[f"gW{i}"], d0, d1)
        tq = _edge_kernel()(*hsq, row_m, row_t, col_m, col_t)
        post_args = (*tq, d0, d1, params[f"gb{i}"], params[f"bng{i}"],
                     params[f"bnb{i}"], pt)
        if i < NUM_LAYERS - 1:
            h, p = _tc_call(
                _post_body,
                [jax.ShapeDtypeStruct((N, D), jnp.float32),
                 jax.ShapeDtypeStruct((G, D), jnp.float32)])(*post_args)
        else:
            p = _tc_call(
                _post_last_body,
                jax.ShapeDtypeStruct((G, D), jnp.float32))(*post_args)
        pooled.append(p)

    out = _tc_call(_final_body, jax.ShapeDtypeStruct((G, OUT), jnp.float32))(
        pt, x, *pooled,
        *(params[f"fcW{i}"] for i in range(NUM_LAYERS + 1)),
        *(params[f"fcb{i}"] for i in range(NUM_LAYERS + 1)))
    return out
